# Initial kernel scaffold; baseline (speedup 1.0000x reference)
import jax
import jax.numpy as jnp
from jax.experimental import pallas as pl


def kernel(x, pos_emb):
    raise NotImplementedError("write your pallas kernel here")


# speedup vs baseline: 3.2081x; 3.2081x over previous
import jax
import jax.numpy as jnp
from jax.experimental import pallas as pl


def _copy_block(src_ref, dst_ref):
    dst_ref[...] = src_ref[...]


def kernel(x, pos_emb):
    del x
    m, d = pos_emb.shape
    blk = 1024
    return pl.pallas_call(
        _copy_block,
        grid=(m // blk,),
        in_specs=[pl.BlockSpec((blk, d), lambda i: (i, 0))],
        out_specs=pl.BlockSpec((blk, d), lambda i: (i, 0)),
        out_shape=jax.ShapeDtypeStruct((m, d), pos_emb.dtype),
    )(pos_emb)
